# baseline (speedup 1.0000x reference)
import jax
import jax.numpy as jnp
from jax import lax
from jax.experimental import pallas as pl
from jax.experimental.pallas import tpu as pltpu
from jax.experimental.pallas import tpu_sc as plsc

_B, _S, _D = 4, 8192, 1024
_M = 2048
_NC, _NS = 2, 16
_NW = _NC * _NS
_RPW = (_B * _M) // _NW
_K = 16
_NCHUNK = _RPW // _K
_NBUF = 4
_NSLOT = 3
_P = 2
_WPB = _M // _RPW


def _body(x_hbm, idx_hbm, out_hbm, shared, idx_v,
          buf0, buf1, buf2, buf3,
          gsem0, gsem1, gsem2, gsem3, csem0, csem1, csem2, csem3,
          dsem0, dsem1, dsem2):
    wid = lax.axis_index("s") * _NC + lax.axis_index("c")
    sid = lax.axis_index("s")
    base = wid * _RPW
    off = (wid // _WPB) * _S
    pltpu.sync_copy(idx_hbm.at[pl.ds(base, _RPW)], idx_v)
    bufs = (buf0, buf1, buf2, buf3)
    gsems = (gsem0, gsem1, gsem2, gsem3)
    csems = (csem0, csem1, csem2, csem3)
    dsems = (dsem0, dsem1, dsem2)

    def start_gather(c):
        iv = idx_v[pl.ds(c * _K, _K)] + off
        pltpu.async_copy(x_hbm.at[iv], bufs[c % _NBUF], gsems[c % _NBUF])

    def wait_gather(c):
        pltpu.make_async_copy(x_hbm.at[pl.ds(0, _K)], bufs[c % _NBUF],
                              gsems[c % _NBUF]).wait()

    def start_cross(c):
        pltpu.async_copy(bufs[c % _NBUF], shared.at[pl.ds((c % _NSLOT) * _NS * _K + sid * _K, _K)],
                         csems[c % _NBUF])

    def wait_cross(c):
        pltpu.make_async_copy(bufs[c % _NBUF], shared.at[pl.ds((c % _NSLOT) * _NS * _K + sid * _K, _K)],
                              csems[c % _NBUF]).wait()

    def start_drain(c):
        pltpu.async_copy(shared.at[pl.ds((c % _NSLOT) * _NS * _K + sid * _K, _K)],
                         out_hbm.at[pl.ds(base + c * _K, _K)], dsems[c % _NSLOT])

    def wait_drain(c):
        pltpu.make_async_copy(shared.at[pl.ds((c % _NSLOT) * _NS * _K + sid * _K, _K)],
                              out_hbm.at[pl.ds(base + c * _K, _K)],
                              dsems[c % _NSLOT]).wait()

    for c0 in range(_P):
        start_gather(c0)
    for c in range(_NCHUNK):
        if c >= 1:
            wait_cross(c - 1)
            start_drain(c - 1)
        wait_gather(c)
        if c >= _NSLOT:
            wait_drain(c - _NSLOT)
        start_cross(c)
        if c + _P < _NCHUNK:
            start_gather(c + _P)
    wait_cross(_NCHUNK - 1)
    start_drain(_NCHUNK - 1)
    for c in range(_NCHUNK - _NSLOT, _NCHUNK):
        wait_drain(c)


@jax.jit
def _gather_flat(xf, idxf):
    mesh = plsc.VectorSubcoreMesh(core_axis_name="c", subcore_axis_name="s")
    f = pl.kernel(
        _body,
        mesh=mesh,
        out_type=jax.ShapeDtypeStruct((_B * _M, _D), jnp.float32),
        scratch_types=[
            pltpu.VMEM_SHARED((_NSLOT * _NS * _K, _D), jnp.float32),
            pltpu.VMEM((_RPW,), jnp.int32),
            pltpu.VMEM((_K, _D), jnp.float32),
            pltpu.VMEM((_K, _D), jnp.float32),
            pltpu.VMEM((_K, _D), jnp.float32),
            pltpu.VMEM((_K, _D), jnp.float32),
        ] + [pltpu.SemaphoreType.DMA] * 11,
    )
    return f(xf, idxf)


def kernel(x, indices_to_keep):
    idxf = indices_to_keep.astype(jnp.int32).reshape(_B * _M)
    xf = x.reshape(_B * _S, _D)
    out = _gather_flat(xf, idxf)
    return out.reshape(_B, _M, _D)
